# Initial kernel scaffold; baseline (speedup 1.0000x reference)
import jax
import jax.numpy as jnp
from jax.experimental import pallas as pl


def kernel(features, edge_index, W1, b1, W2, b2):
    raise NotImplementedError("write your pallas kernel here")


# speedup vs baseline: 5.4601x; 5.4601x over previous
import functools

import jax
import jax.numpy as jnp
from jax import lax
from jax.experimental import pallas as pl
from jax.experimental.pallas import tpu as pltpu
from jax.experimental.pallas import tpu_sc as plsc

N = 10000
E = 320000
D_IN = 128
D_HID = 128
N_CLS = 16

NC = 2
NS = 16
NW = NC * NS

N_PAD = 10240
ROWS_PER_TILE = N_PAD // NS

CH = 128
EPW = E // NW
NCHUNK = 79
EPW_P = NCHUNK * CH

_MESH = plsc.VectorSubcoreMesh(
    core_axis_name="c", subcore_axis_name="s", num_cores=NC, num_subcores=NS
)


def _worker_edge_base(k):
    c = lax.axis_index("c")
    s = lax.axis_index("s")
    w = c * NS + s
    return pl.multiple_of(w * EPW_P + k * CH, CH)


def _deg_body(src_hbm, dst_hbm, onehot_hbm, zeros_hbm, out_hbm,
              idx_s, idx_d, rows_s, rows_d, shared):
    c = lax.axis_index("c")
    s = lax.axis_index("s")
    sl = pl.ds(s * ROWS_PER_TILE, ROWS_PER_TILE)
    pltpu.sync_copy(zeros_hbm.at[sl], shared.at[sl])
    pltpu.sync_copy(onehot_hbm.at[0], rows_s)
    pltpu.sync_copy(onehot_hbm.at[1], rows_d)
    plsc.subcore_barrier()

    @pl.loop(0, NCHUNK)
    def _(k):
        base = _worker_edge_base(k)
        pltpu.sync_copy(src_hbm.at[pl.ds(base, CH)], idx_s)
        pltpu.sync_copy(dst_hbm.at[pl.ds(base, CH)], idx_d)
        pltpu.sync_copy(rows_s, shared.at[idx_s], add=True)
        pltpu.sync_copy(rows_d, shared.at[idx_d], add=True)

    plsc.subcore_barrier()
    pltpu.sync_copy(shared.at[sl], out_hbm.at[c, sl])


def _sc_degrees(src_p, dst_p, onehot, zeros2):
    k = pl.kernel(
        _deg_body,
        out_type=jax.ShapeDtypeStruct((NC, N_PAD, 2), jnp.float32),
        mesh=_MESH,
        scratch_types=[
            pltpu.VMEM((CH,), jnp.int32),
            pltpu.VMEM((CH,), jnp.int32),
            pltpu.VMEM((CH, 2), jnp.float32),
            pltpu.VMEM((CH, 2), jnp.float32),
            pltpu.VMEM_SHARED((N_PAD, 2), jnp.float32),
        ],
    )
    return k(src_p, dst_p, onehot, zeros2)


def _prop_body(h_hbm, src_hbm, dst_hbm, zeros_hbm, out_hbm,
               idx_s, idx_d, rows, shared):
    c = lax.axis_index("c")
    s = lax.axis_index("s")
    sl = pl.ds(s * ROWS_PER_TILE, ROWS_PER_TILE)
    pltpu.sync_copy(zeros_hbm.at[sl], shared.at[sl])
    plsc.subcore_barrier()

    @pl.loop(0, NCHUNK)
    def _(k):
        base = _worker_edge_base(k)
        pltpu.sync_copy(src_hbm.at[pl.ds(base, CH)], idx_s)
        pltpu.sync_copy(dst_hbm.at[pl.ds(base, CH)], idx_d)
        pltpu.sync_copy(h_hbm.at[idx_s], rows)
        pltpu.sync_copy(rows, shared.at[idx_d], add=True)

    plsc.subcore_barrier()
    pltpu.sync_copy(shared.at[sl], out_hbm.at[c, sl])


def _sc_propagate(h, src_p, dst_p, zeros_nd):
    d = h.shape[1]
    k = pl.kernel(
        _prop_body,
        out_type=jax.ShapeDtypeStruct((NC, N_PAD, d), jnp.float32),
        mesh=_MESH,
        compiler_params=pltpu.CompilerParams(use_tc_tiling_on_sc=(d % 128 == 0)),
        scratch_types=[
            pltpu.VMEM((CH,), jnp.int32),
            pltpu.VMEM((CH,), jnp.int32),
            pltpu.VMEM((CH, d), jnp.float32),
            pltpu.VMEM_SHARED((N_PAD, d), jnp.float32),
        ],
    )
    return k(h, src_p, dst_p, zeros_nd)


_BR = 1024
_HI = jax.lax.Precision.HIGHEST


def _mm_body(x_ref, w_ref, o_ref):
    o_ref[...] = lax.dot_general(
        x_ref[...], w_ref[...], (((1,), (0,)), ((), ())),
        precision=_HI, preferred_element_type=jnp.float32)


def _tc_matmul(x, w):
    n, d = x.shape
    dout = w.shape[1]
    return pl.pallas_call(
        _mm_body,
        grid=(n // _BR,),
        in_specs=[
            pl.BlockSpec((_BR, d), lambda i: (i, 0)),
            pl.BlockSpec((d, dout), lambda i: (0, 0)),
        ],
        out_specs=pl.BlockSpec((_BR, dout), lambda i: (i, 0)),
        out_shape=jax.ShapeDtypeStruct((n, dout), jnp.float32),
    )(x, w)


def _scale_body(xw_ref, cnt_ref, h_ref, dout_ref, din_ref):
    deg = cnt_ref[0] + cnt_ref[1]
    dout_s = lax.rsqrt(jnp.maximum(deg[:, 0:1], 1.0))
    din_s = lax.rsqrt(jnp.maximum(deg[:, 1:2], 1.0))
    h_ref[...] = xw_ref[...] * dout_s
    dout_ref[...] = dout_s
    din_ref[...] = din_s


def _tc_scale(xw, cnt):
    return pl.pallas_call(
        _scale_body,
        grid=(N_PAD // _BR,),
        in_specs=[
            pl.BlockSpec((_BR, D_HID), lambda i: (i, 0)),
            pl.BlockSpec((NC, _BR, 2), lambda i: (0, i, 0)),
        ],
        out_specs=[
            pl.BlockSpec((_BR, D_HID), lambda i: (i, 0)),
            pl.BlockSpec((_BR, 1), lambda i: (i, 0)),
            pl.BlockSpec((_BR, 1), lambda i: (i, 0)),
        ],
        out_shape=[
            jax.ShapeDtypeStruct((N_PAD, D_HID), jnp.float32),
            jax.ShapeDtypeStruct((N_PAD, 1), jnp.float32),
            jax.ShapeDtypeStruct((N_PAD, 1), jnp.float32),
        ],
    )(xw, cnt)


def _mid_body(p_ref, din_ref, dout_ref, b1_ref, w2_ref, o_ref):
    agg = (p_ref[0] + p_ref[1]) * din_ref[...]
    h1 = jnp.maximum(agg + b1_ref[...], 0.0)
    t2 = lax.dot_general(h1, w2_ref[...], (((1,), (0,)), ((), ())),
                         precision=_HI, preferred_element_type=jnp.float32)
    o_ref[...] = t2 * dout_ref[...]


def _tc_mid(p, din_s, dout_s, b1, w2):
    return pl.pallas_call(
        _mid_body,
        grid=(N_PAD // _BR,),
        in_specs=[
            pl.BlockSpec((NC, _BR, D_HID), lambda i: (0, i, 0)),
            pl.BlockSpec((_BR, 1), lambda i: (i, 0)),
            pl.BlockSpec((_BR, 1), lambda i: (i, 0)),
            pl.BlockSpec((1, D_HID), lambda i: (0, 0)),
            pl.BlockSpec((D_HID, N_CLS), lambda i: (0, 0)),
        ],
        out_specs=pl.BlockSpec((_BR, N_CLS), lambda i: (i, 0)),
        out_shape=jax.ShapeDtypeStruct((N_PAD, N_CLS), jnp.float32),
    )(p, din_s, dout_s, b1, w2)


def _fin_body(q_ref, din_ref, b2_ref, o_ref):
    o_ref[...] = (q_ref[0] + q_ref[1]) * din_ref[...] + b2_ref[...]


def _tc_final(q, din_s, b2):
    return pl.pallas_call(
        _fin_body,
        grid=(N_PAD // _BR,),
        in_specs=[
            pl.BlockSpec((NC, _BR, N_CLS), lambda i: (0, i, 0)),
            pl.BlockSpec((_BR, 1), lambda i: (i, 0)),
            pl.BlockSpec((1, N_CLS), lambda i: (0, 0)),
        ],
        out_specs=pl.BlockSpec((_BR, N_CLS), lambda i: (i, 0)),
        out_shape=jax.ShapeDtypeStruct((N_PAD, N_CLS), jnp.float32),
    )(q, din_s, b2)


def kernel(features, edge_index, W1, b1, W2, b2):
    f32 = jnp.float32
    x = jnp.zeros((N_PAD, D_IN), f32).at[:N].set(features)

    src = edge_index[0].reshape(NW, EPW)
    dst = edge_index[1].reshape(NW, EPW)
    pad = jnp.full((NW, EPW_P - EPW), N, jnp.int32)
    src_p = jnp.concatenate([src, pad], axis=1).reshape(-1)
    dst_p = jnp.concatenate([dst, pad], axis=1).reshape(-1)

    onehot = jnp.zeros((2, CH, 2), f32).at[0, :, 0].set(1.0).at[1, :, 1].set(1.0)
    zeros2 = jnp.zeros((N_PAD, 2), f32)
    zeros128 = jnp.zeros((N_PAD, D_HID), f32)
    zeros16 = jnp.zeros((N_PAD, N_CLS), f32)

    cnt = _sc_degrees(src_p, dst_p, onehot, zeros2)
    xw = _tc_matmul(x, W1)
    h, dout_s, din_s = _tc_scale(xw, cnt)
    p = _sc_propagate(h, src_p, dst_p, zeros128)
    t2 = _tc_mid(p, din_s, dout_s, b1.reshape(1, D_HID), W2)
    q = _sc_propagate(t2, src_p, dst_p, zeros16)
    out = _tc_final(q, din_s, b2.reshape(1, N_CLS))
    return out[:N]


# speedup vs baseline: 5.8720x; 1.0754x over previous
import jax
import jax.numpy as jnp
from jax import lax
from jax.experimental import pallas as pl
from jax.experimental.pallas import tpu as pltpu
from jax.experimental.pallas import tpu_sc as plsc

N = 10000
E = 320000
D_IN = 128
D_HID = 128
N_CLS = 16

NC = 2
NS = 16
NW = NC * NS

N_PAD = 10240
ROWS_PER_TILE = N_PAD // NS

CH = 128
EPW = E // NW
NCHUNK = 80
EPW_P = NCHUNK * CH

_MESH = plsc.VectorSubcoreMesh(
    core_axis_name="c", subcore_axis_name="s", num_cores=NC, num_subcores=NS
)


def _deg_body(src_hbm, dst_hbm, onehot_hbm, zeros_hbm, out_hbm,
              idx_s, idx_d, rows_s, rows_d, shared):
    c = lax.axis_index("c")
    s = lax.axis_index("s")
    w = c * NS + s
    sl = pl.ds(s * ROWS_PER_TILE, ROWS_PER_TILE)
    pltpu.sync_copy(zeros_hbm.at[sl], shared.at[sl])
    pltpu.sync_copy(onehot_hbm.at[0], rows_s)
    pltpu.sync_copy(onehot_hbm.at[1], rows_d)
    plsc.subcore_barrier()

    @pl.loop(0, NCHUNK)
    def _(k):
        pltpu.sync_copy(src_hbm.at[w, k], idx_s)
        pltpu.sync_copy(dst_hbm.at[w, k], idx_d)
        pltpu.sync_copy(rows_s, shared.at[idx_s], add=True)
        pltpu.sync_copy(rows_d, shared.at[idx_d], add=True)

    plsc.subcore_barrier()
    pltpu.sync_copy(shared.at[sl], out_hbm.at[c, sl])


def _sc_degrees(src_p, dst_p, onehot, zeros2):
    k = pl.kernel(
        _deg_body,
        out_type=jax.ShapeDtypeStruct((NC, N_PAD, 2), jnp.float32),
        mesh=_MESH,
        scratch_types=[
            pltpu.VMEM((CH,), jnp.int32),
            pltpu.VMEM((CH,), jnp.int32),
            pltpu.VMEM((CH, 2), jnp.float32),
            pltpu.VMEM((CH, 2), jnp.float32),
            pltpu.VMEM_SHARED((N_PAD, 2), jnp.float32),
        ],
    )
    return k(src_p, dst_p, onehot, zeros2)


def _prop_body(h_hbm, src_hbm, dst_hbm, zeros_hbm, out_hbm,
               idxs, idxd, r0, r1, si, sd, sg, shared):
    rows = [r0, r1]
    c = lax.axis_index("c")
    s = lax.axis_index("s")
    w = c * NS + s
    sl = pl.ds(s * ROWS_PER_TILE, ROWS_PER_TILE)
    pltpu.sync_copy(zeros_hbm.at[sl], shared.at[sl])
    plsc.subcore_barrier()

    def issue_idx(k):
        q = k % 4
        pltpu.async_copy(src_hbm.at[w, k], idxs.at[q], si.at[q])
        pltpu.async_copy(dst_hbm.at[w, k], idxd.at[q], sd.at[q])

    def wait_idx_s(k):
        q = k % 4
        pltpu.make_async_copy(src_hbm.at[w, k], idxs.at[q], si.at[q]).wait()

    def wait_idx_d(k):
        q = k % 4
        pltpu.make_async_copy(dst_hbm.at[w, k], idxd.at[q], sd.at[q]).wait()

    def issue_g(k, b):
        pltpu.async_copy(h_hbm.at[idxs.at[k % 4]], rows[b], sg.at[b])

    def wait_g(k, b):
        pltpu.make_async_copy(h_hbm.at[idxs.at[k % 4]], rows[b], sg.at[b]).wait()

    def sync_s(k, b):
        pltpu.sync_copy(rows[b], shared.at[idxd.at[k % 4]], add=True)

    for j in range(4):
        issue_idx(j)
    wait_idx_s(0)
    issue_g(0, 0)
    wait_idx_s(1)
    issue_g(1, 1)

    @pl.loop(0, NCHUNK - 4, step=4)
    def _(k0):
        for b4 in range(4):
            k = k0 + b4
            b = b4 % 2
            wait_g(k, b)
            wait_idx_d(k)
            sync_s(k, b)
            issue_idx(k + 4)
            wait_idx_s(k + 2)
            issue_g(k + 2, b)

    for j in range(NCHUNK - 4, NCHUNK):
        b = j % 2
        wait_g(j, b)
        wait_idx_d(j)
        sync_s(j, b)
        if j + 2 < NCHUNK:
            wait_idx_s(j + 2)
            issue_g(j + 2, b)

    plsc.subcore_barrier()
    pltpu.sync_copy(shared.at[sl], out_hbm.at[c, sl])


def _sc_propagate(h, src_p, dst_p, zeros_nd):
    d = h.shape[1]
    k = pl.kernel(
        _prop_body,
        out_type=jax.ShapeDtypeStruct((NC, N_PAD, d), jnp.float32),
        mesh=_MESH,
        compiler_params=pltpu.CompilerParams(use_tc_tiling_on_sc=(d % 128 == 0)),
        scratch_types=[
            pltpu.VMEM((4, CH), jnp.int32),
            pltpu.VMEM((4, CH), jnp.int32),
            pltpu.VMEM((CH, d), jnp.float32),
            pltpu.VMEM((CH, d), jnp.float32),
            pltpu.SemaphoreType.DMA((4,)),
            pltpu.SemaphoreType.DMA((4,)),
            pltpu.SemaphoreType.DMA((2,)),
            pltpu.VMEM_SHARED((N_PAD, d), jnp.float32),
        ],
    )
    return k(h, src_p, dst_p, zeros_nd)


_BR = 1024
_HI = jax.lax.Precision.HIGHEST


def _mm_body(x_ref, w_ref, o_ref):
    o_ref[...] = lax.dot_general(
        x_ref[...], w_ref[...], (((1,), (0,)), ((), ())),
        precision=_HI, preferred_element_type=jnp.float32)


def _tc_matmul(x, w):
    n, d = x.shape
    dout = w.shape[1]
    return pl.pallas_call(
        _mm_body,
        grid=(n // _BR,),
        in_specs=[
            pl.BlockSpec((_BR, d), lambda i: (i, 0)),
            pl.BlockSpec((d, dout), lambda i: (0, 0)),
        ],
        out_specs=pl.BlockSpec((_BR, dout), lambda i: (i, 0)),
        out_shape=jax.ShapeDtypeStruct((n, dout), jnp.float32),
    )(x, w)


def _scale_body(xw_ref, cnt_ref, h_ref, dout_ref, din_ref):
    deg = cnt_ref[0] + cnt_ref[1]
    dout_s = lax.rsqrt(jnp.maximum(deg[:, 0:1], 1.0))
    din_s = lax.rsqrt(jnp.maximum(deg[:, 1:2], 1.0))
    h_ref[...] = xw_ref[...] * dout_s
    dout_ref[...] = dout_s
    din_ref[...] = din_s


def _tc_scale(xw, cnt):
    return pl.pallas_call(
        _scale_body,
        grid=(N_PAD // _BR,),
        in_specs=[
            pl.BlockSpec((_BR, D_HID), lambda i: (i, 0)),
            pl.BlockSpec((NC, _BR, 2), lambda i: (0, i, 0)),
        ],
        out_specs=[
            pl.BlockSpec((_BR, D_HID), lambda i: (i, 0)),
            pl.BlockSpec((_BR, 1), lambda i: (i, 0)),
            pl.BlockSpec((_BR, 1), lambda i: (i, 0)),
        ],
        out_shape=[
            jax.ShapeDtypeStruct((N_PAD, D_HID), jnp.float32),
            jax.ShapeDtypeStruct((N_PAD, 1), jnp.float32),
            jax.ShapeDtypeStruct((N_PAD, 1), jnp.float32),
        ],
    )(xw, cnt)


def _mid_body(p_ref, din_ref, dout_ref, b1_ref, w2_ref, o_ref):
    agg = (p_ref[0] + p_ref[1]) * din_ref[...]
    h1 = jnp.maximum(agg + b1_ref[...], 0.0)
    t2 = lax.dot_general(h1, w2_ref[...], (((1,), (0,)), ((), ())),
                         precision=_HI, preferred_element_type=jnp.float32)
    o_ref[...] = t2 * dout_ref[...]


def _tc_mid(p, din_s, dout_s, b1, w2):
    return pl.pallas_call(
        _mid_body,
        grid=(N_PAD // _BR,),
        in_specs=[
            pl.BlockSpec((NC, _BR, D_HID), lambda i: (0, i, 0)),
            pl.BlockSpec((_BR, 1), lambda i: (i, 0)),
            pl.BlockSpec((_BR, 1), lambda i: (i, 0)),
            pl.BlockSpec((1, D_HID), lambda i: (0, 0)),
            pl.BlockSpec((D_HID, N_CLS), lambda i: (0, 0)),
        ],
        out_specs=pl.BlockSpec((_BR, N_CLS), lambda i: (i, 0)),
        out_shape=jax.ShapeDtypeStruct((N_PAD, N_CLS), jnp.float32),
    )(p, din_s, dout_s, b1, w2)


def _fin_body(q_ref, din_ref, b2_ref, o_ref):
    o_ref[...] = (q_ref[0] + q_ref[1]) * din_ref[...] + b2_ref[...]


def _tc_final(q, din_s, b2):
    return pl.pallas_call(
        _fin_body,
        grid=(N_PAD // _BR,),
        in_specs=[
            pl.BlockSpec((NC, _BR, N_CLS), lambda i: (0, i, 0)),
            pl.BlockSpec((_BR, 1), lambda i: (i, 0)),
            pl.BlockSpec((1, N_CLS), lambda i: (0, 0)),
        ],
        out_specs=pl.BlockSpec((_BR, N_CLS), lambda i: (i, 0)),
        out_shape=jax.ShapeDtypeStruct((N_PAD, N_CLS), jnp.float32),
    )(q, din_s, b2)


def kernel(features, edge_index, W1, b1, W2, b2):
    f32 = jnp.float32
    x = jnp.zeros((N_PAD, D_IN), f32).at[:N].set(features)

    src = edge_index[0].reshape(NW, EPW)
    dst = edge_index[1].reshape(NW, EPW)
    pad = jnp.full((NW, EPW_P - EPW), N, jnp.int32)
    src_p = jnp.concatenate([src, pad], axis=1).reshape(NW, NCHUNK, CH)
    dst_p = jnp.concatenate([dst, pad], axis=1).reshape(NW, NCHUNK, CH)

    onehot = jnp.zeros((2, CH, 2), f32).at[0, :, 0].set(1.0).at[1, :, 1].set(1.0)
    zeros2 = jnp.zeros((N_PAD, 2), f32)
    zeros128 = jnp.zeros((N_PAD, D_HID), f32)
    zeros16 = jnp.zeros((N_PAD, N_CLS), f32)

    cnt = _sc_degrees(src_p, dst_p, onehot, zeros2)
    xw = _tc_matmul(x, W1)
    h, dout_s, din_s = _tc_scale(xw, cnt)
    p = _sc_propagate(h, src_p, dst_p, zeros128)
    t2 = _tc_mid(p, din_s, dout_s, b1.reshape(1, D_HID), W2)
    q = _sc_propagate(t2, src_p, dst_p, zeros16)
    out = _tc_final(q, din_s, b2.reshape(1, N_CLS))
    return out[:N]


# speedup vs baseline: 9.4718x; 1.6130x over previous
import jax
import jax.numpy as jnp
from jax import lax
from jax.experimental import pallas as pl
from jax.experimental.pallas import tpu as pltpu
from jax.experimental.pallas import tpu_sc as plsc

N = 10000
E = 320000
D_IN = 128
D_HID = 128
N_CLS = 16

NC = 2
NS = 16
NW = NC * NS

N_PAD = 10240
ROWS_PER_TILE = N_PAD // NS

CH = 128
EPW = E // NW
NCHUNK = 80
EPW_P = NCHUNK * CH

_MESH = plsc.VectorSubcoreMesh(
    core_axis_name="c", subcore_axis_name="s", num_cores=NC, num_subcores=NS
)


def _deg_body(src_hbm, dst_hbm, onehot_hbm, zeros_hbm, out_hbm,
              idx_s, idx_d, rows_s, rows_d, shared):
    c = lax.axis_index("c")
    s = lax.axis_index("s")
    w = c * NS + s
    sl = pl.ds(s * ROWS_PER_TILE, ROWS_PER_TILE)
    pltpu.sync_copy(zeros_hbm.at[sl], shared.at[sl])
    pltpu.sync_copy(onehot_hbm.at[0], rows_s)
    pltpu.sync_copy(onehot_hbm.at[1], rows_d)
    plsc.subcore_barrier()

    @pl.loop(0, NCHUNK)
    def _(k):
        pltpu.sync_copy(src_hbm.at[w, k], idx_s)
        pltpu.sync_copy(dst_hbm.at[w, k], idx_d)
        pltpu.sync_copy(rows_s, shared.at[idx_s], add=True)
        pltpu.sync_copy(rows_d, shared.at[idx_d], add=True)

    plsc.subcore_barrier()
    pltpu.sync_copy(shared.at[sl], out_hbm.at[c, sl])


def _sc_degrees(src_p, dst_p, onehot, zeros2):
    k = pl.kernel(
        _deg_body,
        out_type=jax.ShapeDtypeStruct((NC, N_PAD, 2), jnp.float32),
        mesh=_MESH,
        scratch_types=[
            pltpu.VMEM((CH,), jnp.int32),
            pltpu.VMEM((CH,), jnp.int32),
            pltpu.VMEM((CH, 2), jnp.float32),
            pltpu.VMEM((CH, 2), jnp.float32),
            pltpu.VMEM_SHARED((N_PAD, 2), jnp.float32),
        ],
    )
    return k(src_p, dst_p, onehot, zeros2)


def _prop_body(h_hbm, src_hbm, dst_hbm, zeros_hbm, out_hbm,
               idxs, idxd, r0, r1, si, sd, sg, tbl, shared):
    rows = [r0, r1]
    c = lax.axis_index("c")
    s = lax.axis_index("s")
    w = c * NS + s
    sl = pl.ds(s * ROWS_PER_TILE, ROWS_PER_TILE)
    pltpu.sync_copy(zeros_hbm.at[sl], shared.at[sl])
    pltpu.sync_copy(h_hbm.at[sl], tbl.at[sl])
    plsc.subcore_barrier()

    def issue_idx(k):
        q = k % 4
        pltpu.async_copy(src_hbm.at[w, k], idxs.at[q], si.at[q])
        pltpu.async_copy(dst_hbm.at[w, k], idxd.at[q], sd.at[q])

    def wait_idx_s(k):
        q = k % 4
        pltpu.make_async_copy(src_hbm.at[w, k], idxs.at[q], si.at[q]).wait()

    def wait_idx_d(k):
        q = k % 4
        pltpu.make_async_copy(dst_hbm.at[w, k], idxd.at[q], sd.at[q]).wait()

    def issue_g(k, b):
        pltpu.async_copy(tbl.at[idxs.at[k % 4]], rows[b], sg.at[b])

    def wait_g(k, b):
        pltpu.make_async_copy(tbl.at[idxs.at[k % 4]], rows[b], sg.at[b]).wait()

    def sync_s(k, b):
        pltpu.sync_copy(rows[b], shared.at[idxd.at[k % 4]], add=True)

    for j in range(4):
        issue_idx(j)
    wait_idx_s(0)
    issue_g(0, 0)
    wait_idx_s(1)
    issue_g(1, 1)

    @pl.loop(0, NCHUNK - 4, step=4)
    def _(k0):
        for b4 in range(4):
            k = k0 + b4
            b = b4 % 2
            wait_g(k, b)
            wait_idx_d(k)
            sync_s(k, b)
            issue_idx(k + 4)
            wait_idx_s(k + 2)
            issue_g(k + 2, b)

    for j in range(NCHUNK - 4, NCHUNK):
        b = j % 2
        wait_g(j, b)
        wait_idx_d(j)
        sync_s(j, b)
        if j + 2 < NCHUNK:
            wait_idx_s(j + 2)
            issue_g(j + 2, b)

    plsc.subcore_barrier()
    pltpu.sync_copy(shared.at[sl], out_hbm.at[c, sl])


def _sc_propagate(h, src_p, dst_p, zeros_nd):
    d = h.shape[1]
    k = pl.kernel(
        _prop_body,
        out_type=jax.ShapeDtypeStruct((NC, N_PAD, d), jnp.float32),
        mesh=_MESH,
        compiler_params=pltpu.CompilerParams(use_tc_tiling_on_sc=(d % 128 == 0)),
        scratch_types=[
            pltpu.VMEM((4, CH), jnp.int32),
            pltpu.VMEM((4, CH), jnp.int32),
            pltpu.VMEM((CH, d), jnp.float32),
            pltpu.VMEM((CH, d), jnp.float32),
            pltpu.SemaphoreType.DMA((4,)),
            pltpu.SemaphoreType.DMA((4,)),
            pltpu.SemaphoreType.DMA((2,)),
            pltpu.VMEM_SHARED((N_PAD, d), jnp.float32),
            pltpu.VMEM_SHARED((N_PAD, d), jnp.float32),
        ],
    )
    return k(h, src_p, dst_p, zeros_nd)


_BR = 1024
_HI = jax.lax.Precision.HIGHEST


def _mm_body(x_ref, w_ref, o_ref):
    o_ref[...] = lax.dot_general(
        x_ref[...], w_ref[...], (((1,), (0,)), ((), ())),
        precision=_HI, preferred_element_type=jnp.float32)


def _tc_matmul(x, w):
    n, d = x.shape
    dout = w.shape[1]
    return pl.pallas_call(
        _mm_body,
        grid=(n // _BR,),
        in_specs=[
            pl.BlockSpec((_BR, d), lambda i: (i, 0)),
            pl.BlockSpec((d, dout), lambda i: (0, 0)),
        ],
        out_specs=pl.BlockSpec((_BR, dout), lambda i: (i, 0)),
        out_shape=jax.ShapeDtypeStruct((n, dout), jnp.float32),
    )(x, w)


def _scale_body(xw_ref, cnt_ref, hlo_ref, hhi_ref, dout_ref, din_ref):
    deg = cnt_ref[0] + cnt_ref[1]
    dout_s = lax.rsqrt(jnp.maximum(deg[:, 0:1], 1.0))
    din_s = lax.rsqrt(jnp.maximum(deg[:, 1:2], 1.0))
    h = xw_ref[...] * dout_s
    hlo_ref[...] = h[:, :D_HID // 2]
    hhi_ref[...] = h[:, D_HID // 2:]
    dout_ref[...] = dout_s
    din_ref[...] = din_s


def _tc_scale(xw, cnt):
    return pl.pallas_call(
        _scale_body,
        grid=(N_PAD // _BR,),
        in_specs=[
            pl.BlockSpec((_BR, D_HID), lambda i: (i, 0)),
            pl.BlockSpec((NC, _BR, 2), lambda i: (0, i, 0)),
        ],
        out_specs=[
            pl.BlockSpec((_BR, D_HID // 2), lambda i: (i, 0)),
            pl.BlockSpec((_BR, D_HID // 2), lambda i: (i, 0)),
            pl.BlockSpec((_BR, 1), lambda i: (i, 0)),
            pl.BlockSpec((_BR, 1), lambda i: (i, 0)),
        ],
        out_shape=[
            jax.ShapeDtypeStruct((N_PAD, D_HID // 2), jnp.float32),
            jax.ShapeDtypeStruct((N_PAD, D_HID // 2), jnp.float32),
            jax.ShapeDtypeStruct((N_PAD, 1), jnp.float32),
            jax.ShapeDtypeStruct((N_PAD, 1), jnp.float32),
        ],
    )(xw, cnt)


def _mid_body(plo_ref, phi_ref, din_ref, dout_ref, b1_ref, w2_ref, o_ref):
    agg = jnp.concatenate(
        [plo_ref[0] + plo_ref[1], phi_ref[0] + phi_ref[1]], axis=1)
    agg = agg * din_ref[...]
    h1 = jnp.maximum(agg + b1_ref[...], 0.0)
    t2 = lax.dot_general(h1, w2_ref[...], (((1,), (0,)), ((), ())),
                         precision=_HI, preferred_element_type=jnp.float32)
    o_ref[...] = t2 * dout_ref[...]


def _tc_mid(p_lo, p_hi, din_s, dout_s, b1, w2):
    return pl.pallas_call(
        _mid_body,
        grid=(N_PAD // _BR,),
        in_specs=[
            pl.BlockSpec((NC, _BR, D_HID // 2), lambda i: (0, i, 0)),
            pl.BlockSpec((NC, _BR, D_HID // 2), lambda i: (0, i, 0)),
            pl.BlockSpec((_BR, 1), lambda i: (i, 0)),
            pl.BlockSpec((_BR, 1), lambda i: (i, 0)),
            pl.BlockSpec((1, D_HID), lambda i: (0, 0)),
            pl.BlockSpec((D_HID, N_CLS), lambda i: (0, 0)),
        ],
        out_specs=pl.BlockSpec((_BR, N_CLS), lambda i: (i, 0)),
        out_shape=jax.ShapeDtypeStruct((N_PAD, N_CLS), jnp.float32),
    )(p_lo, p_hi, din_s, dout_s, b1, w2)


def _fin_body(q_ref, din_ref, b2_ref, o_ref):
    o_ref[...] = (q_ref[0] + q_ref[1]) * din_ref[...] + b2_ref[...]


def _tc_final(q, din_s, b2):
    return pl.pallas_call(
        _fin_body,
        grid=(N_PAD // _BR,),
        in_specs=[
            pl.BlockSpec((NC, _BR, N_CLS), lambda i: (0, i, 0)),
            pl.BlockSpec((_BR, 1), lambda i: (i, 0)),
            pl.BlockSpec((1, N_CLS), lambda i: (0, 0)),
        ],
        out_specs=pl.BlockSpec((_BR, N_CLS), lambda i: (i, 0)),
        out_shape=jax.ShapeDtypeStruct((N_PAD, N_CLS), jnp.float32),
    )(q, din_s, b2)


def kernel(features, edge_index, W1, b1, W2, b2):
    f32 = jnp.float32
    x = jnp.zeros((N_PAD, D_IN), f32).at[:N].set(features)

    src = edge_index[0].reshape(NW, EPW)
    dst = edge_index[1].reshape(NW, EPW)
    pad = jnp.full((NW, EPW_P - EPW), N, jnp.int32)
    src_p = jnp.concatenate([src, pad], axis=1).reshape(NW, NCHUNK, CH)
    dst_p = jnp.concatenate([dst, pad], axis=1).reshape(NW, NCHUNK, CH)

    onehot = jnp.zeros((2, CH, 2), f32).at[0, :, 0].set(1.0).at[1, :, 1].set(1.0)
    zeros2 = jnp.zeros((N_PAD, 2), f32)
    zeros64 = jnp.zeros((N_PAD, D_HID // 2), f32)
    zeros16 = jnp.zeros((N_PAD, N_CLS), f32)

    cnt = _sc_degrees(src_p, dst_p, onehot, zeros2)
    xw = _tc_matmul(x, W1)
    h_lo, h_hi, dout_s, din_s = _tc_scale(xw, cnt)
    p_lo = _sc_propagate(h_lo, src_p, dst_p, zeros64)
    p_hi = _sc_propagate(h_hi, src_p, dst_p, zeros64)
    t2 = _tc_mid(p_lo, p_hi, din_s, dout_s, b1.reshape(1, D_HID), W2)
    q = _sc_propagate(t2, src_p, dst_p, zeros16)
    out = _tc_final(q, din_s, b2.reshape(1, N_CLS))
    return out[:N]


# speedup vs baseline: 10.1082x; 1.0672x over previous
import jax
import jax.numpy as jnp
from jax import lax
from jax.experimental import pallas as pl
from jax.experimental.pallas import tpu as pltpu
from jax.experimental.pallas import tpu_sc as plsc

N = 10000
E = 320000
D_IN = 128
D_HID = 128
N_CLS = 16

NC = 2
NS = 16
NW = NC * NS

N_PAD = 10240
ROWS_PER_TILE = N_PAD // NS

CH = 128
EPW = E // NW
NCHUNK = 80
EPW_P = NCHUNK * CH

_MESH = plsc.VectorSubcoreMesh(
    core_axis_name="c", subcore_axis_name="s", num_cores=NC, num_subcores=NS
)


def _deg_body(src_hbm, dst_hbm, onehot_hbm, zeros_hbm, out_hbm,
              idx_s, idx_d, rows_s, rows_d, shared):
    c = lax.axis_index("c")
    s = lax.axis_index("s")
    w = c * NS + s
    sl = pl.ds(s * ROWS_PER_TILE, ROWS_PER_TILE)
    pltpu.sync_copy(zeros_hbm.at[sl], shared.at[sl])
    pltpu.sync_copy(onehot_hbm.at[0], rows_s)
    pltpu.sync_copy(onehot_hbm.at[1], rows_d)
    plsc.subcore_barrier()

    @pl.loop(0, NCHUNK)
    def _(k):
        pltpu.sync_copy(src_hbm.at[w, k], idx_s)
        pltpu.sync_copy(dst_hbm.at[w, k], idx_d)
        pltpu.sync_copy(rows_s, shared.at[idx_s], add=True)
        pltpu.sync_copy(rows_d, shared.at[idx_d], add=True)

    plsc.subcore_barrier()
    pltpu.sync_copy(shared.at[sl], out_hbm.at[c, sl])


def _sc_degrees(src_p, dst_p, onehot, zeros2):
    k = pl.kernel(
        _deg_body,
        out_type=jax.ShapeDtypeStruct((NC, N_PAD, 2), jnp.float32),
        mesh=_MESH,
        scratch_types=[
            pltpu.VMEM((CH,), jnp.int32),
            pltpu.VMEM((CH,), jnp.int32),
            pltpu.VMEM((CH, 2), jnp.float32),
            pltpu.VMEM((CH, 2), jnp.float32),
            pltpu.VMEM_SHARED((N_PAD, 2), jnp.float32),
        ],
    )
    return k(src_p, dst_p, onehot, zeros2)


def _prop_body(h_hbm, src_hbm, dst_hbm, zeros_hbm, out_hbm,
               idxs, idxd, r0, r1, r2, r3, si, sd, sg, ss, tbl, shared):
    c = lax.axis_index("c")
    s = lax.axis_index("s")
    w = c * NS + s
    sl = pl.ds(s * ROWS_PER_TILE, ROWS_PER_TILE)
    pltpu.sync_copy(zeros_hbm.at[sl], shared.at[sl])
    pltpu.sync_copy(h_hbm.at[sl], tbl.at[sl])
    plsc.subcore_barrier()

    rows = [r0, r1, r2, r3]

    def issue_is(k, q):
        pltpu.async_copy(src_hbm.at[w, k], idxs.at[q], si.at[q])

    def wait_is(k, q):
        pltpu.make_async_copy(src_hbm.at[w, k], idxs.at[q], si.at[q]).wait()

    def issue_id(k, q):
        pltpu.async_copy(dst_hbm.at[w, k], idxd.at[q], sd.at[q])

    def wait_id(k, q):
        pltpu.make_async_copy(dst_hbm.at[w, k], idxd.at[q], sd.at[q]).wait()

    def issue_g(q):
        pltpu.async_copy(tbl.at[idxs.at[q]], rows[q], sg.at[q])

    def wait_g(q):
        pltpu.make_async_copy(tbl.at[idxs.at[q]], rows[q], sg.at[q]).wait()

    def issue_s(q):
        pltpu.async_copy(rows[q], shared.at[idxd.at[q]], ss.at[q], add=True)

    def wait_s(q):
        pltpu.make_async_copy(rows[q], shared.at[idxd.at[q]], ss.at[q]).wait()

    def body(k, q, steady):
        q2 = (q + 2) % 4
        wait_g(q)
        if steady or (isinstance(k, int) and k + 4 < NCHUNK):
            issue_is(k + 4, q)
        wait_id(k, q)
        if steady or (isinstance(k, int) and k >= 2):
            wait_s(q2)
        if steady or (isinstance(k, int) and k + 2 < NCHUNK):
            issue_id(k + 2, q2)
        issue_s(q)
        if steady or (isinstance(k, int) and k + 2 < NCHUNK):
            wait_is(k + 2, q2)
            issue_g(q2)

    for j in range(4):
        issue_is(j, j)
    issue_id(0, 0)
    issue_id(1, 1)
    wait_is(0, 0)
    issue_g(0)
    wait_is(1, 1)
    issue_g(1)

    for j in range(4):
        body(j, j, steady=False)

    @pl.loop(4, NCHUNK - 4, step=4)
    def _(k0):
        for b4 in range(4):
            body(k0 + b4, b4, steady=True)

    for j in range(NCHUNK - 4, NCHUNK):
        body(j, j % 4, steady=False)
    wait_s((NCHUNK - 2) % 4)
    wait_s((NCHUNK - 1) % 4)

    plsc.subcore_barrier()
    pltpu.sync_copy(shared.at[sl], out_hbm.at[c, sl])


def _sc_propagate(h, src_p, dst_p, zeros_nd):
    d = h.shape[1]
    k = pl.kernel(
        _prop_body,
        out_type=jax.ShapeDtypeStruct((NC, N_PAD, d), jnp.float32),
        mesh=_MESH,
        compiler_params=pltpu.CompilerParams(use_tc_tiling_on_sc=(d % 128 == 0)),
        scratch_types=[
            pltpu.VMEM((4, CH), jnp.int32),
            pltpu.VMEM((4, CH), jnp.int32),
            pltpu.VMEM((CH, d), jnp.float32),
            pltpu.VMEM((CH, d), jnp.float32),
            pltpu.VMEM((CH, d), jnp.float32),
            pltpu.VMEM((CH, d), jnp.float32),
            pltpu.SemaphoreType.DMA((4,)),
            pltpu.SemaphoreType.DMA((4,)),
            pltpu.SemaphoreType.DMA((4,)),
            pltpu.SemaphoreType.DMA((4,)),
            pltpu.VMEM_SHARED((N_PAD, d), jnp.float32),
            pltpu.VMEM_SHARED((N_PAD, d), jnp.float32),
        ],
    )
    return k(h, src_p, dst_p, zeros_nd)


_BR = 1024
_HI = jax.lax.Precision.HIGHEST


def _mm_body(x_ref, w_ref, o_ref):
    o_ref[...] = lax.dot_general(
        x_ref[...], w_ref[...], (((1,), (0,)), ((), ())),
        precision=_HI, preferred_element_type=jnp.float32)


def _tc_matmul(x, w):
    n, d = x.shape
    dout = w.shape[1]
    return pl.pallas_call(
        _mm_body,
        grid=(n // _BR,),
        in_specs=[
            pl.BlockSpec((_BR, d), lambda i: (i, 0)),
            pl.BlockSpec((d, dout), lambda i: (0, 0)),
        ],
        out_specs=pl.BlockSpec((_BR, dout), lambda i: (i, 0)),
        out_shape=jax.ShapeDtypeStruct((n, dout), jnp.float32),
    )(x, w)


def _scale_body(xw_ref, cnt_ref, hlo_ref, hhi_ref, dout_ref, din_ref):
    deg = cnt_ref[0] + cnt_ref[1]
    dout_s = lax.rsqrt(jnp.maximum(deg[:, 0:1], 1.0))
    din_s = lax.rsqrt(jnp.maximum(deg[:, 1:2], 1.0))
    h = xw_ref[...] * dout_s
    hlo_ref[...] = h[:, :D_HID // 2]
    hhi_ref[...] = h[:, D_HID // 2:]
    dout_ref[...] = dout_s
    din_ref[...] = din_s


def _tc_scale(xw, cnt):
    return pl.pallas_call(
        _scale_body,
        grid=(N_PAD // _BR,),
        in_specs=[
            pl.BlockSpec((_BR, D_HID), lambda i: (i, 0)),
            pl.BlockSpec((NC, _BR, 2), lambda i: (0, i, 0)),
        ],
        out_specs=[
            pl.BlockSpec((_BR, D_HID // 2), lambda i: (i, 0)),
            pl.BlockSpec((_BR, D_HID // 2), lambda i: (i, 0)),
            pl.BlockSpec((_BR, 1), lambda i: (i, 0)),
            pl.BlockSpec((_BR, 1), lambda i: (i, 0)),
        ],
        out_shape=[
            jax.ShapeDtypeStruct((N_PAD, D_HID // 2), jnp.float32),
            jax.ShapeDtypeStruct((N_PAD, D_HID // 2), jnp.float32),
            jax.ShapeDtypeStruct((N_PAD, 1), jnp.float32),
            jax.ShapeDtypeStruct((N_PAD, 1), jnp.float32),
        ],
    )(xw, cnt)


def _mid_body(plo_ref, phi_ref, din_ref, dout_ref, b1_ref, w2_ref, o_ref):
    agg = jnp.concatenate(
        [plo_ref[0] + plo_ref[1], phi_ref[0] + phi_ref[1]], axis=1)
    agg = agg * din_ref[...]
    h1 = jnp.maximum(agg + b1_ref[...], 0.0)
    t2 = lax.dot_general(h1, w2_ref[...], (((1,), (0,)), ((), ())),
                         precision=_HI, preferred_element_type=jnp.float32)
    o_ref[...] = t2 * dout_ref[...]


def _tc_mid(p_lo, p_hi, din_s, dout_s, b1, w2):
    return pl.pallas_call(
        _mid_body,
        grid=(N_PAD // _BR,),
        in_specs=[
            pl.BlockSpec((NC, _BR, D_HID // 2), lambda i: (0, i, 0)),
            pl.BlockSpec((NC, _BR, D_HID // 2), lambda i: (0, i, 0)),
            pl.BlockSpec((_BR, 1), lambda i: (i, 0)),
            pl.BlockSpec((_BR, 1), lambda i: (i, 0)),
            pl.BlockSpec((1, D_HID), lambda i: (0, 0)),
            pl.BlockSpec((D_HID, N_CLS), lambda i: (0, 0)),
        ],
        out_specs=pl.BlockSpec((_BR, N_CLS), lambda i: (i, 0)),
        out_shape=jax.ShapeDtypeStruct((N_PAD, N_CLS), jnp.float32),
    )(p_lo, p_hi, din_s, dout_s, b1, w2)


def _fin_body(q_ref, din_ref, b2_ref, o_ref):
    o_ref[...] = (q_ref[0] + q_ref[1]) * din_ref[...] + b2_ref[...]


def _tc_final(q, din_s, b2):
    return pl.pallas_call(
        _fin_body,
        grid=(N_PAD // _BR,),
        in_specs=[
            pl.BlockSpec((NC, _BR, N_CLS), lambda i: (0, i, 0)),
            pl.BlockSpec((_BR, 1), lambda i: (i, 0)),
            pl.BlockSpec((1, N_CLS), lambda i: (0, 0)),
        ],
        out_specs=pl.BlockSpec((_BR, N_CLS), lambda i: (i, 0)),
        out_shape=jax.ShapeDtypeStruct((N_PAD, N_CLS), jnp.float32),
    )(q, din_s, b2)


def kernel(features, edge_index, W1, b1, W2, b2):
    f32 = jnp.float32
    x = jnp.zeros((N_PAD, D_IN), f32).at[:N].set(features)

    src = edge_index[0].reshape(NW, EPW)
    dst = edge_index[1].reshape(NW, EPW)
    pad = jnp.full((NW, EPW_P - EPW), N, jnp.int32)
    src_p = jnp.concatenate([src, pad], axis=1).reshape(NW, NCHUNK, CH)
    dst_p = jnp.concatenate([dst, pad], axis=1).reshape(NW, NCHUNK, CH)

    onehot = jnp.zeros((2, CH, 2), f32).at[0, :, 0].set(1.0).at[1, :, 1].set(1.0)
    zeros2 = jnp.zeros((N_PAD, 2), f32)
    zeros64 = jnp.zeros((N_PAD, D_HID // 2), f32)
    zeros16 = jnp.zeros((N_PAD, N_CLS), f32)

    cnt = _sc_degrees(src_p, dst_p, onehot, zeros2)
    xw = _tc_matmul(x, W1)
    h_lo, h_hi, dout_s, din_s = _tc_scale(xw, cnt)
    p_lo = _sc_propagate(h_lo, src_p, dst_p, zeros64)
    p_hi = _sc_propagate(h_hi, src_p, dst_p, zeros64)
    t2 = _tc_mid(p_lo, p_hi, din_s, dout_s, b1.reshape(1, D_HID), W2)
    q = _sc_propagate(t2, src_p, dst_p, zeros16)
    out = _tc_final(q, din_s, b2.reshape(1, N_CLS))
    return out[:N]


# speedup vs baseline: 12.1415x; 1.2012x over previous
import jax
import jax.numpy as jnp
from jax import lax
from jax.experimental import pallas as pl
from jax.experimental.pallas import tpu as pltpu
from jax.experimental.pallas import tpu_sc as plsc

N = 10000
E = 320000
D_IN = 128
D_HID = 128
N_CLS = 16

NC = 2
NS = 16
NW = NC * NS

N_PAD = 10240
ROWS_PER_TILE = N_PAD // NS

CH = 128
EPW = E // NW
NCHUNK = 80
EPW_P = NCHUNK * CH

_MESH = plsc.VectorSubcoreMesh(
    core_axis_name="c", subcore_axis_name="s", num_cores=NC, num_subcores=NS
)


def _deg_body(src_hbm, dst_hbm, onehot_hbm, zeros_hbm, out_hbm,
              idxs, idxd, rows_s, rows_d, si, sd, scs, scd, shared):
    c = lax.axis_index("c")
    s = lax.axis_index("s")
    w = c * NS + s
    sl = pl.ds(s * ROWS_PER_TILE, ROWS_PER_TILE)
    pltpu.sync_copy(zeros_hbm.at[sl], shared.at[sl])
    pltpu.sync_copy(onehot_hbm.at[0], rows_s)
    pltpu.sync_copy(onehot_hbm.at[1], rows_d)
    plsc.subcore_barrier()

    def issue_is(k, q):
        pltpu.async_copy(src_hbm.at[w, k], idxs.at[q], si.at[q])

    def wait_is(k, q):
        pltpu.make_async_copy(src_hbm.at[w, k], idxs.at[q], si.at[q]).wait()

    def issue_id(k, q):
        pltpu.async_copy(dst_hbm.at[w, k], idxd.at[q], sd.at[q])

    def wait_id(k, q):
        pltpu.make_async_copy(dst_hbm.at[w, k], idxd.at[q], sd.at[q]).wait()

    def issue_ss(q):
        pltpu.async_copy(rows_s, shared.at[idxs.at[q]], scs.at[q], add=True)

    def wait_ss(q):
        pltpu.make_async_copy(rows_s, shared.at[idxs.at[q]], scs.at[q]).wait()

    def issue_sd(q):
        pltpu.async_copy(rows_d, shared.at[idxd.at[q]], scd.at[q], add=True)

    def wait_sd(q):
        pltpu.make_async_copy(rows_d, shared.at[idxd.at[q]], scd.at[q]).wait()

    def body(k, q, steady):
        q2 = (q + 2) % 4
        wait_is(k, q)
        wait_id(k, q)
        if steady or (isinstance(k, int) and k >= 2):
            wait_ss(q2)
            wait_sd(q2)
        issue_ss(q)
        issue_sd(q)
        if steady or (isinstance(k, int) and k + 2 < NCHUNK):
            issue_is(k + 2, q2)
            issue_id(k + 2, q2)

    issue_is(0, 0)
    issue_id(0, 0)
    issue_is(1, 1)
    issue_id(1, 1)
    for j in range(2):
        body(j, j, steady=False)

    @pl.loop(2, NCHUNK - 2, step=4)
    def _(k0):
        for b4 in range(4):
            body(k0 + b4, (2 + b4) % 4, steady=True)

    for j in range(NCHUNK - 2, NCHUNK):
        body(j, j % 4, steady=False)
    wait_ss((NCHUNK - 2) % 4)
    wait_sd((NCHUNK - 2) % 4)
    wait_ss((NCHUNK - 1) % 4)
    wait_sd((NCHUNK - 1) % 4)

    plsc.subcore_barrier()
    pltpu.sync_copy(shared.at[sl], out_hbm.at[c, sl])


def _sc_degrees(src_p, dst_p, onehot, zeros2):
    k = pl.kernel(
        _deg_body,
        out_type=jax.ShapeDtypeStruct((NC, N_PAD, 2), jnp.float32),
        mesh=_MESH,
        scratch_types=[
            pltpu.VMEM((4, CH), jnp.int32),
            pltpu.VMEM((4, CH), jnp.int32),
            pltpu.VMEM((CH, 2), jnp.float32),
            pltpu.VMEM((CH, 2), jnp.float32),
            pltpu.SemaphoreType.DMA((4,)),
            pltpu.SemaphoreType.DMA((4,)),
            pltpu.SemaphoreType.DMA((4,)),
            pltpu.SemaphoreType.DMA((4,)),
            pltpu.VMEM_SHARED((N_PAD, 2), jnp.float32),
        ],
    )
    return k(src_p, dst_p, onehot, zeros2)


def _prop_body(h_hbm, src_hbm, dst_hbm, zeros_hbm, out_hbm,
               idxs, idxd, r0, r1, r2, r3, si, sd, sg, ss, tbl, shared):
    c = lax.axis_index("c")
    s = lax.axis_index("s")
    w = c * NS + s
    sl = pl.ds(s * ROWS_PER_TILE, ROWS_PER_TILE)
    pltpu.sync_copy(zeros_hbm.at[sl], shared.at[sl])
    pltpu.sync_copy(h_hbm.at[sl], tbl.at[sl])
    plsc.subcore_barrier()

    rows = [r0, r1, r2, r3]

    def issue_is(k, q):
        pltpu.async_copy(src_hbm.at[w, k], idxs.at[q], si.at[q])

    def wait_is(k, q):
        pltpu.make_async_copy(src_hbm.at[w, k], idxs.at[q], si.at[q]).wait()

    def issue_id(k, q):
        pltpu.async_copy(dst_hbm.at[w, k], idxd.at[q], sd.at[q])

    def wait_id(k, q):
        pltpu.make_async_copy(dst_hbm.at[w, k], idxd.at[q], sd.at[q]).wait()

    def issue_g(q):
        pltpu.async_copy(tbl.at[idxs.at[q]], rows[q], sg.at[q])

    def wait_g(q):
        pltpu.make_async_copy(tbl.at[idxs.at[q]], rows[q], sg.at[q]).wait()

    def issue_s(q):
        pltpu.async_copy(rows[q], shared.at[idxd.at[q]], ss.at[q], add=True)

    def wait_s(q):
        pltpu.make_async_copy(rows[q], shared.at[idxd.at[q]], ss.at[q]).wait()

    def body(k, q, steady):
        q2 = (q + 2) % 4
        wait_g(q)
        if steady or (isinstance(k, int) and k + 4 < NCHUNK):
            issue_is(k + 4, q)
        wait_id(k, q)
        if steady or (isinstance(k, int) and k >= 2):
            wait_s(q2)
        if steady or (isinstance(k, int) and k + 2 < NCHUNK):
            issue_id(k + 2, q2)
        issue_s(q)
        if steady or (isinstance(k, int) and k + 2 < NCHUNK):
            wait_is(k + 2, q2)
            issue_g(q2)

    for j in range(4):
        issue_is(j, j)
    issue_id(0, 0)
    issue_id(1, 1)
    wait_is(0, 0)
    issue_g(0)
    wait_is(1, 1)
    issue_g(1)

    for j in range(4):
        body(j, j, steady=False)

    @pl.loop(4, NCHUNK - 4, step=4)
    def _(k0):
        for b4 in range(4):
            body(k0 + b4, b4, steady=True)

    for j in range(NCHUNK - 4, NCHUNK):
        body(j, j % 4, steady=False)
    wait_s((NCHUNK - 2) % 4)
    wait_s((NCHUNK - 1) % 4)

    plsc.subcore_barrier()
    pltpu.sync_copy(shared.at[sl], out_hbm.at[c, sl])


def _sc_propagate(h, src_p, dst_p, zeros_nd):
    d = h.shape[1]
    k = pl.kernel(
        _prop_body,
        out_type=jax.ShapeDtypeStruct((NC, N_PAD, d), jnp.float32),
        mesh=_MESH,
        compiler_params=pltpu.CompilerParams(use_tc_tiling_on_sc=(d % 128 == 0)),
        scratch_types=[
            pltpu.VMEM((4, CH), jnp.int32),
            pltpu.VMEM((4, CH), jnp.int32),
            pltpu.VMEM((CH, d), jnp.float32),
            pltpu.VMEM((CH, d), jnp.float32),
            pltpu.VMEM((CH, d), jnp.float32),
            pltpu.VMEM((CH, d), jnp.float32),
            pltpu.SemaphoreType.DMA((4,)),
            pltpu.SemaphoreType.DMA((4,)),
            pltpu.SemaphoreType.DMA((4,)),
            pltpu.SemaphoreType.DMA((4,)),
            pltpu.VMEM_SHARED((N_PAD, d), jnp.float32),
            pltpu.VMEM_SHARED((N_PAD, d), jnp.float32),
        ],
    )
    return k(h, src_p, dst_p, zeros_nd)


_BR = 1024
_HI = jax.lax.Precision.HIGHEST


def _mm_body(x_ref, w_ref, o_ref):
    o_ref[...] = lax.dot_general(
        x_ref[...], w_ref[...], (((1,), (0,)), ((), ())),
        precision=_HI, preferred_element_type=jnp.float32)


def _tc_matmul(x, w):
    n, d = x.shape
    dout = w.shape[1]
    return pl.pallas_call(
        _mm_body,
        grid=(n // _BR,),
        in_specs=[
            pl.BlockSpec((_BR, d), lambda i: (i, 0)),
            pl.BlockSpec((d, dout), lambda i: (0, 0)),
        ],
        out_specs=pl.BlockSpec((_BR, dout), lambda i: (i, 0)),
        out_shape=jax.ShapeDtypeStruct((n, dout), jnp.float32),
    )(x, w)


def _scale_body(xw_ref, cnt_ref, hlo_ref, hhi_ref, dout_ref, din_ref):
    deg = cnt_ref[0] + cnt_ref[1]
    dout_s = lax.rsqrt(jnp.maximum(deg[:, 0:1], 1.0))
    din_s = lax.rsqrt(jnp.maximum(deg[:, 1:2], 1.0))
    h = xw_ref[...] * dout_s
    hlo_ref[...] = h[:, :D_HID // 2]
    hhi_ref[...] = h[:, D_HID // 2:]
    dout_ref[...] = dout_s
    din_ref[...] = din_s


def _tc_scale(xw, cnt):
    return pl.pallas_call(
        _scale_body,
        grid=(N_PAD // _BR,),
        in_specs=[
            pl.BlockSpec((_BR, D_HID), lambda i: (i, 0)),
            pl.BlockSpec((NC, _BR, 2), lambda i: (0, i, 0)),
        ],
        out_specs=[
            pl.BlockSpec((_BR, D_HID // 2), lambda i: (i, 0)),
            pl.BlockSpec((_BR, D_HID // 2), lambda i: (i, 0)),
            pl.BlockSpec((_BR, 1), lambda i: (i, 0)),
            pl.BlockSpec((_BR, 1), lambda i: (i, 0)),
        ],
        out_shape=[
            jax.ShapeDtypeStruct((N_PAD, D_HID // 2), jnp.float32),
            jax.ShapeDtypeStruct((N_PAD, D_HID // 2), jnp.float32),
            jax.ShapeDtypeStruct((N_PAD, 1), jnp.float32),
            jax.ShapeDtypeStruct((N_PAD, 1), jnp.float32),
        ],
    )(xw, cnt)


def _mid_body(plo_ref, phi_ref, din_ref, dout_ref, b1_ref, w2_ref, o_ref):
    agg = jnp.concatenate(
        [plo_ref[0] + plo_ref[1], phi_ref[0] + phi_ref[1]], axis=1)
    agg = agg * din_ref[...]
    h1 = jnp.maximum(agg + b1_ref[...], 0.0)
    t2 = lax.dot_general(h1, w2_ref[...], (((1,), (0,)), ((), ())),
                         precision=_HI, preferred_element_type=jnp.float32)
    o_ref[...] = t2 * dout_ref[...]


def _tc_mid(p_lo, p_hi, din_s, dout_s, b1, w2):
    return pl.pallas_call(
        _mid_body,
        grid=(N_PAD // _BR,),
        in_specs=[
            pl.BlockSpec((NC, _BR, D_HID // 2), lambda i: (0, i, 0)),
            pl.BlockSpec((NC, _BR, D_HID // 2), lambda i: (0, i, 0)),
            pl.BlockSpec((_BR, 1), lambda i: (i, 0)),
            pl.BlockSpec((_BR, 1), lambda i: (i, 0)),
            pl.BlockSpec((1, D_HID), lambda i: (0, 0)),
            pl.BlockSpec((D_HID, N_CLS), lambda i: (0, 0)),
        ],
        out_specs=pl.BlockSpec((_BR, N_CLS), lambda i: (i, 0)),
        out_shape=jax.ShapeDtypeStruct((N_PAD, N_CLS), jnp.float32),
    )(p_lo, p_hi, din_s, dout_s, b1, w2)


def _fin_body(q_ref, din_ref, b2_ref, o_ref):
    o_ref[...] = (q_ref[0] + q_ref[1]) * din_ref[...] + b2_ref[...]


def _tc_final(q, din_s, b2):
    return pl.pallas_call(
        _fin_body,
        grid=(N_PAD // _BR,),
        in_specs=[
            pl.BlockSpec((NC, _BR, N_CLS), lambda i: (0, i, 0)),
            pl.BlockSpec((_BR, 1), lambda i: (i, 0)),
            pl.BlockSpec((1, N_CLS), lambda i: (0, 0)),
        ],
        out_specs=pl.BlockSpec((_BR, N_CLS), lambda i: (i, 0)),
        out_shape=jax.ShapeDtypeStruct((N_PAD, N_CLS), jnp.float32),
    )(q, din_s, b2)


def kernel(features, edge_index, W1, b1, W2, b2):
    f32 = jnp.float32
    x = jnp.zeros((N_PAD, D_IN), f32).at[:N].set(features)

    src = edge_index[0].reshape(NW, EPW)
    dst = edge_index[1].reshape(NW, EPW)
    pad = jnp.full((NW, EPW_P - EPW), N, jnp.int32)
    src_p = jnp.concatenate([src, pad], axis=1).reshape(NW, NCHUNK, CH)
    dst_p = jnp.concatenate([dst, pad], axis=1).reshape(NW, NCHUNK, CH)

    onehot = jnp.zeros((2, CH, 2), f32).at[0, :, 0].set(1.0).at[1, :, 1].set(1.0)
    zeros2 = jnp.zeros((N_PAD, 2), f32)
    zeros64 = jnp.zeros((N_PAD, D_HID // 2), f32)
    zeros16 = jnp.zeros((N_PAD, N_CLS), f32)

    cnt = _sc_degrees(src_p, dst_p, onehot, zeros2)
    xw = _tc_matmul(x, W1)
    h_lo, h_hi, dout_s, din_s = _tc_scale(xw, cnt)
    p_lo = _sc_propagate(h_lo, src_p, dst_p, zeros64)
    p_hi = _sc_propagate(h_hi, src_p, dst_p, zeros64)
    t2 = _tc_mid(p_lo, p_hi, din_s, dout_s, b1.reshape(1, D_HID), W2)
    q = _sc_propagate(t2, src_p, dst_p, zeros16)
    out = _tc_final(q, din_s, b2.reshape(1, N_CLS))
    return out[:N]


# speedup vs baseline: 13.1939x; 1.0867x over previous
import jax
import jax.numpy as jnp
from jax import lax
from jax.experimental import pallas as pl
from jax.experimental.pallas import tpu as pltpu
from jax.experimental.pallas import tpu_sc as plsc

N = 10000
E = 320000
D_IN = 128
D_HID = 128
N_CLS = 16

NC = 2
NS = 16
NW = NC * NS

N_PAD = 10240
ROWS_PER_TILE = N_PAD // NS

CH = 128
EPW = E // NW
NCHUNK = 80
EPW_P = NCHUNK * CH

_MESH = plsc.VectorSubcoreMesh(
    core_axis_name="c", subcore_axis_name="s", num_cores=NC, num_subcores=NS
)


def _deg_body(src_hbm, dst_hbm, onehot_hbm, zeros_hbm, out_hbm,
              idxs, idxd, rows_s, rows_d, si, sd, scs, scd, shared):
    c = lax.axis_index("c")
    s = lax.axis_index("s")
    w = c * NS + s
    sl = pl.ds(s * ROWS_PER_TILE, ROWS_PER_TILE)
    pltpu.sync_copy(zeros_hbm.at[sl], shared.at[sl])
    pltpu.sync_copy(onehot_hbm.at[0], rows_s)
    pltpu.sync_copy(onehot_hbm.at[1], rows_d)
    plsc.subcore_barrier()

    def issue_is(k, q):
        pltpu.async_copy(src_hbm.at[w, k], idxs.at[q], si.at[q])

    def wait_is(k, q):
        pltpu.make_async_copy(src_hbm.at[w, k], idxs.at[q], si.at[q]).wait()

    def issue_id(k, q):
        pltpu.async_copy(dst_hbm.at[w, k], idxd.at[q], sd.at[q])

    def wait_id(k, q):
        pltpu.make_async_copy(dst_hbm.at[w, k], idxd.at[q], sd.at[q]).wait()

    def issue_ss(q):
        pltpu.async_copy(rows_s, shared.at[idxs.at[q]], scs.at[q], add=True)

    def wait_ss(q):
        pltpu.make_async_copy(rows_s, shared.at[idxs.at[q]], scs.at[q]).wait()

    def issue_sd(q):
        pltpu.async_copy(rows_d, shared.at[idxd.at[q]], scd.at[q], add=True)

    def wait_sd(q):
        pltpu.make_async_copy(rows_d, shared.at[idxd.at[q]], scd.at[q]).wait()

    def body(k, q, steady):
        q2 = (q + 2) % 4
        wait_is(k, q)
        wait_id(k, q)
        if steady or (isinstance(k, int) and k >= 2):
            wait_ss(q2)
            wait_sd(q2)
        issue_ss(q)
        issue_sd(q)
        if steady or (isinstance(k, int) and k + 2 < NCHUNK):
            issue_is(k + 2, q2)
            issue_id(k + 2, q2)

    issue_is(0, 0)
    issue_id(0, 0)
    issue_is(1, 1)
    issue_id(1, 1)
    for j in range(2):
        body(j, j, steady=False)

    @pl.loop(2, NCHUNK - 2, step=4)
    def _(k0):
        for b4 in range(4):
            body(k0 + b4, (2 + b4) % 4, steady=True)

    for j in range(NCHUNK - 2, NCHUNK):
        body(j, j % 4, steady=False)
    wait_ss((NCHUNK - 2) % 4)
    wait_sd((NCHUNK - 2) % 4)
    wait_ss((NCHUNK - 1) % 4)
    wait_sd((NCHUNK - 1) % 4)

    plsc.subcore_barrier()
    pltpu.sync_copy(shared.at[sl], out_hbm.at[c, sl])


def _sc_degrees(src_p, dst_p, onehot, zeros2):
    k = pl.kernel(
        _deg_body,
        out_type=jax.ShapeDtypeStruct((NC, N_PAD, 2), jnp.float32),
        mesh=_MESH,
        scratch_types=[
            pltpu.VMEM((4, CH), jnp.int32),
            pltpu.VMEM((4, CH), jnp.int32),
            pltpu.VMEM((CH, 2), jnp.float32),
            pltpu.VMEM((CH, 2), jnp.float32),
            pltpu.SemaphoreType.DMA((4,)),
            pltpu.SemaphoreType.DMA((4,)),
            pltpu.SemaphoreType.DMA((4,)),
            pltpu.SemaphoreType.DMA((4,)),
            pltpu.VMEM_SHARED((N_PAD, 2), jnp.float32),
        ],
    )
    return k(src_p, dst_p, onehot, zeros2)


def _make_prop_body(nchunk, merged):
  def _prop_body(h_hbm, src_hbm, dst_hbm, zeros_hbm, out_hbm,
                 idxs, idxd, r0, r1, r2, r3, si, sd, sg, ss, tbl, shared):
    c = lax.axis_index("c")
    s = lax.axis_index("s")
    w = s if merged else c * NS + s
    sl = pl.ds(s * ROWS_PER_TILE, ROWS_PER_TILE)
    pltpu.sync_copy(zeros_hbm.at[sl], shared.at[sl])
    if merged:
        half = D_HID // 2
        pltpu.sync_copy(h_hbm.at[sl, pl.ds(c * half, half)], tbl.at[sl])
    else:
        pltpu.sync_copy(h_hbm.at[sl], tbl.at[sl])
    plsc.subcore_barrier()

    rows = [r0, r1, r2, r3]

    def issue_is(k, q):
        pltpu.async_copy(src_hbm.at[w, k], idxs.at[q], si.at[q])

    def wait_is(k, q):
        pltpu.make_async_copy(src_hbm.at[w, k], idxs.at[q], si.at[q]).wait()

    def issue_id(k, q):
        pltpu.async_copy(dst_hbm.at[w, k], idxd.at[q], sd.at[q])

    def wait_id(k, q):
        pltpu.make_async_copy(dst_hbm.at[w, k], idxd.at[q], sd.at[q]).wait()

    def issue_g(q):
        pltpu.async_copy(tbl.at[idxs.at[q]], rows[q], sg.at[q])

    def wait_g(q):
        pltpu.make_async_copy(tbl.at[idxs.at[q]], rows[q], sg.at[q]).wait()

    def issue_s(q):
        pltpu.async_copy(rows[q], shared.at[idxd.at[q]], ss.at[q], add=True)

    def wait_s(q):
        pltpu.make_async_copy(rows[q], shared.at[idxd.at[q]], ss.at[q]).wait()

    def body(k, q, steady):
        q2 = (q + 2) % 4
        wait_g(q)
        if steady or (isinstance(k, int) and k + 4 < nchunk):
            issue_is(k + 4, q)
        wait_id(k, q)
        if steady or (isinstance(k, int) and k >= 2):
            wait_s(q2)
        if steady or (isinstance(k, int) and k + 2 < nchunk):
            issue_id(k + 2, q2)
        issue_s(q)
        if steady or (isinstance(k, int) and k + 2 < nchunk):
            wait_is(k + 2, q2)
            issue_g(q2)

    for j in range(4):
        issue_is(j, j)
    issue_id(0, 0)
    issue_id(1, 1)
    wait_is(0, 0)
    issue_g(0)
    wait_is(1, 1)
    issue_g(1)

    for j in range(4):
        body(j, j, steady=False)

    @pl.loop(4, nchunk - 4, step=4)
    def _(k0):
        for b4 in range(4):
            body(k0 + b4, b4, steady=True)

    for j in range(nchunk - 4, nchunk):
        body(j, j % 4, steady=False)
    wait_s((nchunk - 2) % 4)
    wait_s((nchunk - 1) % 4)

    plsc.subcore_barrier()
    pltpu.sync_copy(shared.at[sl], out_hbm.at[c, sl])

  return _prop_body


def _sc_propagate(h, src_p, dst_p, zeros_nd, merged=False):
    d = zeros_nd.shape[1]
    nchunk = src_p.shape[1]
    k = pl.kernel(
        _make_prop_body(nchunk, merged),
        out_type=jax.ShapeDtypeStruct((NC, N_PAD, d), jnp.float32),
        mesh=_MESH,
        compiler_params=pltpu.CompilerParams(use_tc_tiling_on_sc=(d % 128 == 0)),
        scratch_types=[
            pltpu.VMEM((4, CH), jnp.int32),
            pltpu.VMEM((4, CH), jnp.int32),
            pltpu.VMEM((CH, d), jnp.float32),
            pltpu.VMEM((CH, d), jnp.float32),
            pltpu.VMEM((CH, d), jnp.float32),
            pltpu.VMEM((CH, d), jnp.float32),
            pltpu.SemaphoreType.DMA((4,)),
            pltpu.SemaphoreType.DMA((4,)),
            pltpu.SemaphoreType.DMA((4,)),
            pltpu.SemaphoreType.DMA((4,)),
            pltpu.VMEM_SHARED((N_PAD, d), jnp.float32),
            pltpu.VMEM_SHARED((N_PAD, d), jnp.float32),
        ],
    )
    return k(h, src_p, dst_p, zeros_nd)


_BR = 1024
_HI = jax.lax.Precision.HIGHEST


def _mm_body(x_ref, w_ref, o_ref):
    o_ref[...] = lax.dot_general(
        x_ref[...], w_ref[...], (((1,), (0,)), ((), ())),
        precision=_HI, preferred_element_type=jnp.float32)


def _tc_matmul(x, w):
    n, d = x.shape
    dout = w.shape[1]
    return pl.pallas_call(
        _mm_body,
        grid=(n // _BR,),
        in_specs=[
            pl.BlockSpec((_BR, d), lambda i: (i, 0)),
            pl.BlockSpec((d, dout), lambda i: (0, 0)),
        ],
        out_specs=pl.BlockSpec((_BR, dout), lambda i: (i, 0)),
        out_shape=jax.ShapeDtypeStruct((n, dout), jnp.float32),
    )(x, w)


def _scale_body(xw_ref, cnt_ref, h_ref, dout_ref, din_ref):
    deg = cnt_ref[0] + cnt_ref[1]
    dout_s = lax.rsqrt(jnp.maximum(deg[:, 0:1], 1.0))
    din_s = lax.rsqrt(jnp.maximum(deg[:, 1:2], 1.0))
    h_ref[...] = xw_ref[...] * dout_s
    dout_ref[...] = dout_s
    din_ref[...] = din_s


def _tc_scale(xw, cnt):
    return pl.pallas_call(
        _scale_body,
        grid=(N_PAD // _BR,),
        in_specs=[
            pl.BlockSpec((_BR, D_HID), lambda i: (i, 0)),
            pl.BlockSpec((NC, _BR, 2), lambda i: (0, i, 0)),
        ],
        out_specs=[
            pl.BlockSpec((_BR, D_HID), lambda i: (i, 0)),
            pl.BlockSpec((_BR, 1), lambda i: (i, 0)),
            pl.BlockSpec((_BR, 1), lambda i: (i, 0)),
        ],
        out_shape=[
            jax.ShapeDtypeStruct((N_PAD, D_HID), jnp.float32),
            jax.ShapeDtypeStruct((N_PAD, 1), jnp.float32),
            jax.ShapeDtypeStruct((N_PAD, 1), jnp.float32),
        ],
    )(xw, cnt)


def _mid_body(p_ref, din_ref, dout_ref, b1_ref, w2_ref, o_ref):
    agg = jnp.concatenate([p_ref[0], p_ref[1]], axis=1)
    agg = agg * din_ref[...]
    h1 = jnp.maximum(agg + b1_ref[...], 0.0)
    t2 = lax.dot_general(h1, w2_ref[...], (((1,), (0,)), ((), ())),
                         precision=_HI, preferred_element_type=jnp.float32)
    o_ref[...] = t2 * dout_ref[...]


def _tc_mid(p, din_s, dout_s, b1, w2):
    return pl.pallas_call(
        _mid_body,
        grid=(N_PAD // _BR,),
        in_specs=[
            pl.BlockSpec((NC, _BR, D_HID // 2), lambda i: (0, i, 0)),
            pl.BlockSpec((_BR, 1), lambda i: (i, 0)),
            pl.BlockSpec((_BR, 1), lambda i: (i, 0)),
            pl.BlockSpec((1, D_HID), lambda i: (0, 0)),
            pl.BlockSpec((D_HID, N_CLS), lambda i: (0, 0)),
        ],
        out_specs=pl.BlockSpec((_BR, N_CLS), lambda i: (i, 0)),
        out_shape=jax.ShapeDtypeStruct((N_PAD, N_CLS), jnp.float32),
    )(p, din_s, dout_s, b1, w2)


def _fin_body(q_ref, din_ref, b2_ref, o_ref):
    o_ref[...] = (q_ref[0] + q_ref[1]) * din_ref[...] + b2_ref[...]


def _tc_final(q, din_s, b2):
    return pl.pallas_call(
        _fin_body,
        grid=(N_PAD // _BR,),
        in_specs=[
            pl.BlockSpec((NC, _BR, N_CLS), lambda i: (0, i, 0)),
            pl.BlockSpec((_BR, 1), lambda i: (i, 0)),
            pl.BlockSpec((1, N_CLS), lambda i: (0, 0)),
        ],
        out_specs=pl.BlockSpec((_BR, N_CLS), lambda i: (i, 0)),
        out_shape=jax.ShapeDtypeStruct((N_PAD, N_CLS), jnp.float32),
    )(q, din_s, b2)


def kernel(features, edge_index, W1, b1, W2, b2):
    f32 = jnp.float32
    x = jnp.zeros((N_PAD, D_IN), f32).at[:N].set(features)

    src = edge_index[0].reshape(NW, EPW)
    dst = edge_index[1].reshape(NW, EPW)
    pad = jnp.full((NW, EPW_P - EPW), N, jnp.int32)
    src_p = jnp.concatenate([src, pad], axis=1).reshape(NW, NCHUNK, CH)
    dst_p = jnp.concatenate([dst, pad], axis=1).reshape(NW, NCHUNK, CH)

    onehot = jnp.zeros((2, CH, 2), f32).at[0, :, 0].set(1.0).at[1, :, 1].set(1.0)
    zeros2 = jnp.zeros((N_PAD, 2), f32)
    zeros64 = jnp.zeros((N_PAD, D_HID // 2), f32)
    zeros16 = jnp.zeros((N_PAD, N_CLS), f32)

    src_m = src_p.reshape(NS, 2 * NCHUNK, CH)
    dst_m = dst_p.reshape(NS, 2 * NCHUNK, CH)

    cnt = _sc_degrees(src_p, dst_p, onehot, zeros2)
    xw = _tc_matmul(x, W1)
    h, dout_s, din_s = _tc_scale(xw, cnt)
    p = _sc_propagate(h, src_m, dst_m, zeros64, merged=True)
    t2 = _tc_mid(p, din_s, dout_s, b1.reshape(1, D_HID), W2)
    q = _sc_propagate(t2, src_p, dst_p, zeros16)
    out = _tc_final(q, din_s, b2.reshape(1, N_CLS))
    return out[:N]


# speedup vs baseline: 13.2114x; 1.0013x over previous
import jax
import jax.numpy as jnp
from jax import lax
from jax.experimental import pallas as pl
from jax.experimental.pallas import tpu as pltpu
from jax.experimental.pallas import tpu_sc as plsc

N = 10000
E = 320000
D_IN = 128
D_HID = 128
N_CLS = 16

NC = 2
NS = 16
NW = NC * NS

N_PAD = 10240
ROWS_PER_TILE = N_PAD // NS

CH = 128
EPW = E // NW
NCHUNK = 80
EPW_P = NCHUNK * CH

def _mesh():
    return plsc.VectorSubcoreMesh(
        core_axis_name="c", subcore_axis_name="s", num_cores=NC, num_subcores=NS
    )


def _deg_body(src_hbm, dst_hbm, onehot_hbm, zeros_hbm, out_hbm,
              idxs, idxd, rows_s, rows_d, si, sd, scs, scd, shared):
    c = lax.axis_index("c")
    s = lax.axis_index("s")
    w = c * NS + s
    sl = pl.ds(s * ROWS_PER_TILE, ROWS_PER_TILE)
    pltpu.sync_copy(zeros_hbm.at[sl], shared.at[sl])
    pltpu.sync_copy(onehot_hbm.at[0], rows_s)
    pltpu.sync_copy(onehot_hbm.at[1], rows_d)
    plsc.subcore_barrier()

    def issue_is(k, q):
        pltpu.async_copy(src_hbm.at[w, k], idxs.at[q], si.at[q])

    def wait_is(k, q):
        pltpu.make_async_copy(src_hbm.at[w, k], idxs.at[q], si.at[q]).wait()

    def issue_id(k, q):
        pltpu.async_copy(dst_hbm.at[w, k], idxd.at[q], sd.at[q])

    def wait_id(k, q):
        pltpu.make_async_copy(dst_hbm.at[w, k], idxd.at[q], sd.at[q]).wait()

    def issue_ss(q):
        pltpu.async_copy(rows_s, shared.at[idxs.at[q]], scs.at[q], add=True)

    def wait_ss(q):
        pltpu.make_async_copy(rows_s, shared.at[idxs.at[q]], scs.at[q]).wait()

    def issue_sd(q):
        pltpu.async_copy(rows_d, shared.at[idxd.at[q]], scd.at[q], add=True)

    def wait_sd(q):
        pltpu.make_async_copy(rows_d, shared.at[idxd.at[q]], scd.at[q]).wait()

    def body(k, q, steady):
        q2 = (q + 2) % 4
        wait_is(k, q)
        wait_id(k, q)
        if steady or (isinstance(k, int) and k >= 2):
            wait_ss(q2)
            wait_sd(q2)
        issue_ss(q)
        issue_sd(q)
        if steady or (isinstance(k, int) and k + 2 < NCHUNK):
            issue_is(k + 2, q2)
            issue_id(k + 2, q2)

    issue_is(0, 0)
    issue_id(0, 0)
    issue_is(1, 1)
    issue_id(1, 1)
    for j in range(2):
        body(j, j, steady=False)

    @pl.loop(2, NCHUNK - 2, step=4)
    def _(k0):
        for b4 in range(4):
            body(k0 + b4, (2 + b4) % 4, steady=True)

    for j in range(NCHUNK - 2, NCHUNK):
        body(j, j % 4, steady=False)
    wait_ss((NCHUNK - 2) % 4)
    wait_sd((NCHUNK - 2) % 4)
    wait_ss((NCHUNK - 1) % 4)
    wait_sd((NCHUNK - 1) % 4)

    plsc.subcore_barrier()
    pltpu.sync_copy(shared.at[sl], out_hbm.at[c, sl])


def _sc_degrees(src_p, dst_p, onehot, zeros2):
    k = pl.kernel(
        _deg_body,
        out_type=jax.ShapeDtypeStruct((NC, N_PAD, 2), jnp.float32),
        mesh=_mesh(),
        scratch_types=[
            pltpu.VMEM((4, CH), jnp.int32),
            pltpu.VMEM((4, CH), jnp.int32),
            pltpu.VMEM((CH, 2), jnp.float32),
            pltpu.VMEM((CH, 2), jnp.float32),
            pltpu.SemaphoreType.DMA((4,)),
            pltpu.SemaphoreType.DMA((4,)),
            pltpu.SemaphoreType.DMA((4,)),
            pltpu.SemaphoreType.DMA((4,)),
            pltpu.VMEM_SHARED((N_PAD, 2), jnp.float32),
        ],
    )
    return k(src_p, dst_p, onehot, zeros2)


def _make_prop_body(nchunk, merged):
  def _prop_body(h_hbm, src_hbm, dst_hbm, zeros_hbm, out_hbm,
                 idxs, idxd, r0, r1, r2, r3, si, sd, sg, ss, tbl, shared):
    c = lax.axis_index("c")
    s = lax.axis_index("s")
    w = s if merged else c * NS + s
    sl = pl.ds(s * ROWS_PER_TILE, ROWS_PER_TILE)
    pltpu.sync_copy(zeros_hbm.at[sl], shared.at[sl])
    if merged:
        half = D_HID // 2
        pltpu.sync_copy(h_hbm.at[sl, pl.ds(c * half, half)], tbl.at[sl])
    else:
        pltpu.sync_copy(h_hbm.at[sl], tbl.at[sl])
    plsc.subcore_barrier()

    rows = [r0, r1, r2, r3]

    def issue_is(k, q):
        pltpu.async_copy(src_hbm.at[w, k], idxs.at[q], si.at[q])

    def wait_is(k, q):
        pltpu.make_async_copy(src_hbm.at[w, k], idxs.at[q], si.at[q]).wait()

    def issue_id(k, q):
        pltpu.async_copy(dst_hbm.at[w, k], idxd.at[q], sd.at[q])

    def wait_id(k, q):
        pltpu.make_async_copy(dst_hbm.at[w, k], idxd.at[q], sd.at[q]).wait()

    def issue_g(q):
        pltpu.async_copy(tbl.at[idxs.at[q]], rows[q], sg.at[q])

    def wait_g(q):
        pltpu.make_async_copy(tbl.at[idxs.at[q]], rows[q], sg.at[q]).wait()

    def issue_s(q):
        pltpu.async_copy(rows[q], shared.at[idxd.at[q]], ss.at[q], add=True)

    def wait_s(q):
        pltpu.make_async_copy(rows[q], shared.at[idxd.at[q]], ss.at[q]).wait()

    def body(k, q, steady):
        q2 = (q + 2) % 4
        wait_g(q)
        if steady or (isinstance(k, int) and k + 4 < nchunk):
            issue_is(k + 4, q)
        wait_id(k, q)
        if steady or (isinstance(k, int) and k >= 2):
            wait_s(q2)
        if steady or (isinstance(k, int) and k + 2 < nchunk):
            issue_id(k + 2, q2)
        issue_s(q)
        if steady or (isinstance(k, int) and k + 2 < nchunk):
            wait_is(k + 2, q2)
            issue_g(q2)

    for j in range(4):
        issue_is(j, j)
    issue_id(0, 0)
    issue_id(1, 1)
    wait_is(0, 0)
    issue_g(0)
    wait_is(1, 1)
    issue_g(1)

    for j in range(4):
        body(j, j, steady=False)

    @pl.loop(4, nchunk - 4, step=4)
    def _(k0):
        for b4 in range(4):
            body(k0 + b4, b4, steady=True)

    for j in range(nchunk - 4, nchunk):
        body(j, j % 4, steady=False)
    wait_s((nchunk - 2) % 4)
    wait_s((nchunk - 1) % 4)

    plsc.subcore_barrier()
    pltpu.sync_copy(shared.at[sl], out_hbm.at[c, sl])

  return _prop_body


def _sc_propagate(h, src_p, dst_p, zeros_nd, merged=False):
    d = zeros_nd.shape[1]
    nchunk = src_p.shape[1]
    k = pl.kernel(
        _make_prop_body(nchunk, merged),
        out_type=jax.ShapeDtypeStruct((NC, N_PAD, d), jnp.float32),
        mesh=_mesh(),
        compiler_params=pltpu.CompilerParams(use_tc_tiling_on_sc=(d % 128 == 0)),
        scratch_types=[
            pltpu.VMEM((4, CH), jnp.int32),
            pltpu.VMEM((4, CH), jnp.int32),
            pltpu.VMEM((CH, d), jnp.float32),
            pltpu.VMEM((CH, d), jnp.float32),
            pltpu.VMEM((CH, d), jnp.float32),
            pltpu.VMEM((CH, d), jnp.float32),
            pltpu.SemaphoreType.DMA((4,)),
            pltpu.SemaphoreType.DMA((4,)),
            pltpu.SemaphoreType.DMA((4,)),
            pltpu.SemaphoreType.DMA((4,)),
            pltpu.VMEM_SHARED((N_PAD, d), jnp.float32),
            pltpu.VMEM_SHARED((N_PAD, d), jnp.float32),
        ],
    )
    return k(h, src_p, dst_p, zeros_nd)


_BR = 1024
_HI = jax.lax.Precision.HIGHEST


def _mm_body(x_ref, w_ref, o_ref):
    o_ref[...] = lax.dot_general(
        x_ref[...], w_ref[...], (((1,), (0,)), ((), ())),
        precision=_HI, preferred_element_type=jnp.float32)


def _tc_matmul(x, w):
    n, d = x.shape
    dout = w.shape[1]
    return pl.pallas_call(
        _mm_body,
        grid=(n // _BR,),
        in_specs=[
            pl.BlockSpec((_BR, d), lambda i: (i, 0)),
            pl.BlockSpec((d, dout), lambda i: (0, 0)),
        ],
        out_specs=pl.BlockSpec((_BR, dout), lambda i: (i, 0)),
        out_shape=jax.ShapeDtypeStruct((n, dout), jnp.float32),
    )(x, w)


def _scale_body(xw_ref, cnt_ref, h_ref, dout_ref, din_ref):
    deg = cnt_ref[0] + cnt_ref[1]
    dout_s = lax.rsqrt(jnp.maximum(deg[:, 0:1], 1.0))
    din_s = lax.rsqrt(jnp.maximum(deg[:, 1:2], 1.0))
    h_ref[...] = xw_ref[...] * dout_s
    dout_ref[...] = dout_s
    din_ref[...] = din_s


def _tc_scale(xw, cnt):
    return pl.pallas_call(
        _scale_body,
        grid=(N_PAD // _BR,),
        in_specs=[
            pl.BlockSpec((_BR, D_HID), lambda i: (i, 0)),
            pl.BlockSpec((NC, _BR, 2), lambda i: (0, i, 0)),
        ],
        out_specs=[
            pl.BlockSpec((_BR, D_HID), lambda i: (i, 0)),
            pl.BlockSpec((_BR, 1), lambda i: (i, 0)),
            pl.BlockSpec((_BR, 1), lambda i: (i, 0)),
        ],
        out_shape=[
            jax.ShapeDtypeStruct((N_PAD, D_HID), jnp.float32),
            jax.ShapeDtypeStruct((N_PAD, 1), jnp.float32),
            jax.ShapeDtypeStruct((N_PAD, 1), jnp.float32),
        ],
    )(xw, cnt)


def _mid_body(p_ref, din_ref, dout_ref, b1_ref, w2_ref, o_ref):
    agg = jnp.concatenate([p_ref[0], p_ref[1]], axis=1)
    agg = agg * din_ref[...]
    h1 = jnp.maximum(agg + b1_ref[...], 0.0)
    t2 = lax.dot_general(h1, w2_ref[...], (((1,), (0,)), ((), ())),
                         precision=_HI, preferred_element_type=jnp.float32)
    o_ref[...] = t2 * dout_ref[...]


def _tc_mid(p, din_s, dout_s, b1, w2):
    return pl.pallas_call(
        _mid_body,
        grid=(N_PAD // _BR,),
        in_specs=[
            pl.BlockSpec((NC, _BR, D_HID // 2), lambda i: (0, i, 0)),
            pl.BlockSpec((_BR, 1), lambda i: (i, 0)),
            pl.BlockSpec((_BR, 1), lambda i: (i, 0)),
            pl.BlockSpec((1, D_HID), lambda i: (0, 0)),
            pl.BlockSpec((D_HID, N_CLS), lambda i: (0, 0)),
        ],
        out_specs=pl.BlockSpec((_BR, N_CLS), lambda i: (i, 0)),
        out_shape=jax.ShapeDtypeStruct((N_PAD, N_CLS), jnp.float32),
    )(p, din_s, dout_s, b1, w2)


def _fin_body(q_ref, din_ref, b2_ref, o_ref):
    o_ref[...] = (q_ref[0] + q_ref[1]) * din_ref[...] + b2_ref[...]


def _tc_final(q, din_s, b2):
    return pl.pallas_call(
        _fin_body,
        grid=(N_PAD // _BR,),
        in_specs=[
            pl.BlockSpec((NC, _BR, N_CLS), lambda i: (0, i, 0)),
            pl.BlockSpec((_BR, 1), lambda i: (i, 0)),
            pl.BlockSpec((1, N_CLS), lambda i: (0, 0)),
        ],
        out_specs=pl.BlockSpec((_BR, N_CLS), lambda i: (i, 0)),
        out_shape=jax.ShapeDtypeStruct((N_PAD, N_CLS), jnp.float32),
    )(q, din_s, b2)


def kernel(features, edge_index, W1, b1, W2, b2):
    f32 = jnp.float32
    x = jnp.zeros((N_PAD, D_IN), f32).at[:N].set(features)

    src = edge_index[0].reshape(NW, EPW)
    dst = edge_index[1].reshape(NW, EPW)
    pad = jnp.full((NW, EPW_P - EPW), N, jnp.int32)
    src_p = jnp.concatenate([src, pad], axis=1).reshape(NW, NCHUNK, CH)
    dst_p = jnp.concatenate([dst, pad], axis=1).reshape(NW, NCHUNK, CH)

    onehot = jnp.zeros((2, CH, 2), f32).at[0, :, 0].set(1.0).at[1, :, 1].set(1.0)
    zeros2 = jnp.zeros((N_PAD, 2), f32)
    zeros64 = jnp.zeros((N_PAD, D_HID // 2), f32)
    zeros16 = jnp.zeros((N_PAD, N_CLS), f32)

    src_m = src_p.reshape(NS, 2 * NCHUNK, CH)
    dst_m = dst_p.reshape(NS, 2 * NCHUNK, CH)

    cnt = _sc_degrees(src_p, dst_p, onehot, zeros2)
    xw = _tc_matmul(x, W1)
    h, dout_s, din_s = _tc_scale(xw, cnt)
    p = _sc_propagate(h, src_m, dst_m, zeros64, merged=True)
    t2 = _tc_mid(p, din_s, dout_s, b1.reshape(1, D_HID), W2)
    q = _sc_propagate(t2, src_p, dst_p, zeros16)
    out = _tc_final(q, din_s, b2.reshape(1, N_CLS))
    return out[:N]
